# baseline (speedup 1.0000x reference)
import functools

import jax
import jax.numpy as jnp
from jax.experimental import pallas as pl
import jax.experimental.pallas.tpu as pltpu

BB = 8192


def _fuzzy_kernel(x_ref, mu_ref, sigma_ref, idx_ref, out_ref, v_ref, c_ref):
    @pl.when(pl.program_id(0) == 0)
    def _prep():
        mu = mu_ref[...]
        sg = sigma_ref[...]
        idx = idx_ref[...]
        m = mu.shape[0]
        iota = jax.lax.broadcasted_iota(jnp.int32, (m,) + idx.shape, 0)
        onehot = (iota == idx[None, :, :]).astype(jnp.float32)
        mu_g = jnp.sum(onehot * mu[:, None, :], axis=0)
        sg_g = jnp.sum(onehot * sg[:, None, :], axis=0)
        sg_g = jnp.maximum(sg_g, 1e-15)
        w = 1.0 / (sg_g * sg_g)
        a = mu_g * w
        v_ref[...] = jnp.concatenate([w, -2.0 * a], axis=1)
        q = mu_g * a
        c_ref[...] = jax.lax.dot_general(
            jnp.ones((1, q.shape[1]), jnp.float32), q,
            (((1,), (1,)), ((), ())),
            preferred_element_type=jnp.float32,
            precision=jax.lax.Precision.HIGHEST,
        )

    x = x_ref[...]
    u = jnp.concatenate([x * x, x], axis=1)
    s = jax.lax.dot_general(
        u, v_ref[...], (((1,), (1,)), ((), ())),
        preferred_element_type=jnp.float32,
        precision=jax.lax.Precision.HIGHEST,
    ) + c_ref[...]
    out_ref[...] = jnp.exp(-0.5 * jnp.maximum(s, 0.0))


@functools.partial(jax.jit, static_argnames=("interpret",))
def kernel(x, mu, sigma, rule_masks, interpret=False):
    b, i = x.shape
    r = rule_masks.shape[0]
    grid = (b // BB,)
    return pl.pallas_call(
        _fuzzy_kernel,
        grid=grid,
        in_specs=[
            pl.BlockSpec((BB, i), lambda j: (j, 0)),
            pl.BlockSpec(mu.shape, lambda j: (0, 0)),
            pl.BlockSpec(sigma.shape, lambda j: (0, 0)),
            pl.BlockSpec(rule_masks.shape, lambda j: (0, 0)),
        ],
        out_specs=pl.BlockSpec((BB, r), lambda j: (j, 0)),
        out_shape=jax.ShapeDtypeStruct((b, r), jnp.float32),
        scratch_shapes=[
            pltpu.VMEM((r, 2 * i), jnp.float32),
            pltpu.VMEM((1, r), jnp.float32),
        ],
        interpret=interpret,
    )(x, mu, sigma, rule_masks.astype(jnp.int32))


# speedup vs baseline: 1.0407x; 1.0407x over previous
import functools

import jax
import jax.numpy as jnp
from jax.experimental import pallas as pl
import jax.experimental.pallas.tpu as pltpu

BB = 4096


def _fuzzy_kernel(x_ref, mu_ref, sigma_ref, idx_ref, out_ref, v_ref, c_ref):
    @pl.when(pl.program_id(0) == 0)
    def _prep():
        mu = mu_ref[...]
        sg = sigma_ref[...]
        idx = idx_ref[...]
        m = mu.shape[0]
        iota = jax.lax.broadcasted_iota(jnp.int32, (m,) + idx.shape, 0)
        onehot = (iota == idx[None, :, :]).astype(jnp.float32)
        mu_g = jnp.sum(onehot * mu[:, None, :], axis=0)
        sg_g = jnp.sum(onehot * sg[:, None, :], axis=0)
        sg_g = jnp.maximum(sg_g, 1e-15)
        w = 1.0 / (sg_g * sg_g)
        a = mu_g * w
        k = -0.72134752044448170368
        v_ref[...] = jnp.concatenate([k * w, (-2.0 * k) * a], axis=1)
        q = mu_g * a
        c_ref[...] = k * jax.lax.dot_general(
            jnp.ones((1, q.shape[1]), jnp.float32), q,
            (((1,), (1,)), ((), ())),
            preferred_element_type=jnp.float32,
            precision=jax.lax.Precision.HIGHEST,
        )

    x = x_ref[...]
    u = jnp.concatenate([x * x, x], axis=1)
    s = jax.lax.dot_general(
        u, v_ref[...], (((1,), (1,)), ((), ())),
        preferred_element_type=jnp.float32,
        precision=jax.lax.Precision.HIGHEST,
    ) + c_ref[...]
    out_ref[...] = jnp.exp2(jnp.minimum(s, 0.0))


@functools.partial(jax.jit, static_argnames=("interpret",))
def kernel(x, mu, sigma, rule_masks, interpret=False):
    b, i = x.shape
    r = rule_masks.shape[0]
    grid = (b // BB,)
    return pl.pallas_call(
        _fuzzy_kernel,
        grid=grid,
        in_specs=[
            pl.BlockSpec((BB, i), lambda j: (j, 0)),
            pl.BlockSpec(mu.shape, lambda j: (0, 0)),
            pl.BlockSpec(sigma.shape, lambda j: (0, 0)),
            pl.BlockSpec(rule_masks.shape, lambda j: (0, 0)),
        ],
        out_specs=pl.BlockSpec((BB, r), lambda j: (j, 0)),
        out_shape=jax.ShapeDtypeStruct((b, r), jnp.float32),
        scratch_shapes=[
            pltpu.VMEM((r, 2 * i), jnp.float32),
            pltpu.VMEM((1, r), jnp.float32),
        ],
        interpret=interpret,
    )(x, mu, sigma, rule_masks.astype(jnp.int32))
